# Initial kernel scaffold; baseline (speedup 1.0000x reference)
import jax
import jax.numpy as jnp
from jax.experimental import pallas as pl


def kernel(x, W_g):
    raise NotImplementedError("write your pallas kernel here")


# speedup vs baseline: 2.2747x; 2.2747x over previous
import functools

import jax
import jax.numpy as jnp
from jax import lax
from jax.experimental import pallas as pl
from jax.experimental.pallas import tpu as pltpu
from jax.experimental.pallas import tpu_sc as plsc

IN_F = 768
NE = 8
BETA_F = 1.0e6
V_IMP_F = 0.1

BT = 2048

SC_CORES = 2
SC_SUBCORES = 16
SC_LANES = 16
NW = SC_CORES * SC_SUBCORES


def _tc_body(x_ref, w_ref, g_ref, laux_ref, imp_ref, *, nb):
    i = pl.program_id(0)
    g = jnp.dot(x_ref[...], w_ref[...], preferred_element_type=jnp.float32)
    g_ref[...] = g
    bs = jnp.sum(g, axis=0, keepdims=True)

    @pl.when(i == 0)
    def _():
        imp_ref[...] = bs

    @pl.when(i != 0)
    def _():
        imp_ref[...] = imp_ref[...] + bs

    @pl.when(i == nb - 1)
    def _():
        imp = imp_ref[...]
        mean = jnp.mean(imp)
        var = jnp.sum((imp - mean) ** 2) * (1.0 / (NE - 1))
        laux_ref[...] = jnp.reshape(V_IMP_F * var / (mean * mean), (1, 1))


def _tc_router(x, wt):
    n = x.shape[0]
    nb = n // BT
    return pl.pallas_call(
        functools.partial(_tc_body, nb=nb),
        grid=(nb,),
        in_specs=[
            pl.BlockSpec((BT, IN_F), lambda i: (i, 0)),
            pl.BlockSpec((IN_F, NE), lambda i: (0, 0)),
        ],
        out_specs=[
            pl.BlockSpec((BT, NE), lambda i: (i, 0)),
            pl.BlockSpec((1, 1), lambda i: (0, 0)),
        ],
        out_shape=[
            jax.ShapeDtypeStruct((n, NE), jnp.float32),
            jax.ShapeDtypeStruct((1, 1), jnp.float32),
        ],
        scratch_shapes=[pltpu.VMEM((1, NE), jnp.float32)],
    )(x, wt)


def _sc_gates(gflat, n):
    per_w = n // NW
    flat_w = per_w * NE
    mesh = plsc.VectorSubcoreMesh(
        core_axis_name="c", subcore_axis_name="s",
        num_cores=SC_CORES, num_subcores=SC_SUBCORES,
    )

    @functools.partial(
        pl.kernel,
        mesh=mesh,
        out_type=jax.ShapeDtypeStruct((n * NE,), jnp.float32),
        scratch_types=[
            pltpu.VMEM((flat_w,), jnp.float32),
            pltpu.VMEM((flat_w,), jnp.float32),
        ],
        compiler_params=pltpu.CompilerParams(needs_layout_passes=False),
    )
    def k(g_hbm, out_hbm, g_v, o_v):
        wid = lax.axis_index("c") * SC_SUBCORES + lax.axis_index("s")
        base = wid * flat_w
        pltpu.sync_copy(g_hbm.at[pl.ds(base, flat_w)], g_v)

        def body(t, carry):
            tbase = t * (SC_LANES * NE) + lax.iota(jnp.int32, SC_LANES) * NE
            vs = [plsc.load_gather(g_v, [tbase + e]) for e in range(NE)]
            best = vs[0]
            bi = jnp.zeros((SC_LANES,), jnp.int32)
            for e in range(1, NE):
                gt = vs[e] > best
                best = jnp.where(gt, vs[e], best)
                bi = jnp.where(gt, jnp.full((SC_LANES,), e, jnp.int32), bi)
            m = jnp.maximum(best, BETA_F)
            e_fill = jnp.exp(BETA_F - m)
            e_top = jnp.exp(best - m)
            inv = 1.0 / ((NE - 1) * e_fill + e_top)
            g_fill = e_fill * inv
            g_top = e_top * inv
            for e in range(NE):
                oe = jnp.where(bi == e, g_top, g_fill)
                plsc.store_scatter(o_v, [tbase + e], oe)
            return carry

        lax.fori_loop(0, per_w // SC_LANES, body, 0)
        pltpu.sync_copy(o_v, out_hbm.at[pl.ds(base, flat_w)])

    return k(gflat)


def kernel(x, W_g):
    wt = W_g.T
    g, laux = _tc_router(x, wt)
    n = x.shape[0]
    gates = jnp.reshape(_sc_gates(jnp.reshape(g, (n * NE,)), n), (n, NE))
    return gates, jnp.reshape(laux, ())


# speedup vs baseline: 2.2941x; 1.0085x over previous
import functools

import jax
import jax.numpy as jnp
from jax import lax
from jax.experimental import pallas as pl
from jax.experimental.pallas import tpu as pltpu
from jax.experimental.pallas import tpu_sc as plsc

IN_F = 768
NE = 8
BETA_F = 1.0e6
V_IMP_F = 0.1

BT = 4096

SC_CORES = 2
SC_SUBCORES = 16
SC_LANES = 16
NW = SC_CORES * SC_SUBCORES


def _tc_body(x_ref, w_ref, g_ref, laux_ref, imp_ref, *, nb):
    i = pl.program_id(0)
    g = jnp.dot(x_ref[...], w_ref[...], preferred_element_type=jnp.float32)
    g_ref[...] = g
    bs = jnp.sum(g, axis=0, keepdims=True)

    @pl.when(i == 0)
    def _():
        imp_ref[...] = bs

    @pl.when(i != 0)
    def _():
        imp_ref[...] = imp_ref[...] + bs

    @pl.when(i == nb - 1)
    def _():
        imp = imp_ref[...]
        mean = jnp.mean(imp)
        var = jnp.sum((imp - mean) ** 2) * (1.0 / (NE - 1))
        laux_ref[...] = jnp.reshape(V_IMP_F * var / (mean * mean), (1, 1))


def _tc_router(x, wt):
    n = x.shape[0]
    nb = n // BT
    return pl.pallas_call(
        functools.partial(_tc_body, nb=nb),
        grid=(nb,),
        in_specs=[
            pl.BlockSpec((BT, IN_F), lambda i: (i, 0)),
            pl.BlockSpec((IN_F, NE), lambda i: (0, 0)),
        ],
        out_specs=[
            pl.BlockSpec((BT, NE), lambda i: (i, 0)),
            pl.BlockSpec((1, 1), lambda i: (0, 0)),
        ],
        out_shape=[
            jax.ShapeDtypeStruct((n, NE), jnp.float32),
            jax.ShapeDtypeStruct((1, 1), jnp.float32),
        ],
        scratch_shapes=[pltpu.VMEM((1, NE), jnp.float32)],
    )(x, wt)


def _sc_gates(gflat, n):
    per_w = n // NW
    flat_w = per_w * NE
    mesh = plsc.VectorSubcoreMesh(
        core_axis_name="c", subcore_axis_name="s",
        num_cores=SC_CORES, num_subcores=SC_SUBCORES,
    )

    @functools.partial(
        pl.kernel,
        mesh=mesh,
        out_type=jax.ShapeDtypeStruct((n * NE,), jnp.float32),
        scratch_types=[
            pltpu.VMEM((flat_w,), jnp.float32),
            pltpu.VMEM((flat_w,), jnp.float32),
        ],
        compiler_params=pltpu.CompilerParams(needs_layout_passes=False),
    )
    def k(g_hbm, out_hbm, g_v, o_v):
        wid = lax.axis_index("c") * SC_SUBCORES + lax.axis_index("s")
        base = wid * flat_w
        pltpu.sync_copy(g_hbm.at[pl.ds(base, flat_w)], g_v)

        def body(t, carry):
            tbase = t * (SC_LANES * NE) + lax.iota(jnp.int32, SC_LANES) * NE
            vs = [plsc.load_gather(g_v, [tbase + e]) for e in range(NE)]
            best = vs[0]
            bi = jnp.zeros((SC_LANES,), jnp.int32)
            for e in range(1, NE):
                gt = vs[e] > best
                best = jnp.where(gt, vs[e], best)
                bi = jnp.where(gt, jnp.full((SC_LANES,), e, jnp.int32), bi)
            m = jnp.maximum(best, BETA_F)
            e_fill = jnp.exp(BETA_F - m)
            e_top = jnp.exp(best - m)
            inv = 1.0 / ((NE - 1) * e_fill + e_top)
            g_fill = e_fill * inv
            g_top = e_top * inv
            for e in range(NE):
                oe = jnp.where(bi == e, g_top, g_fill)
                plsc.store_scatter(o_v, [tbase + e], oe)
            return carry

        lax.fori_loop(0, per_w // SC_LANES, body, 0)
        pltpu.sync_copy(o_v, out_hbm.at[pl.ds(base, flat_w)])

    return k(gflat)


def kernel(x, W_g):
    wt = W_g.T
    g, laux = _tc_router(x, wt)
    n = x.shape[0]
    gates = jnp.reshape(_sc_gates(jnp.reshape(g, (n * NE,)), n), (n, NE))
    return gates, jnp.reshape(laux, ())
